# baseline (speedup 1.0000x reference)
import functools

import jax
import jax.numpy as jnp
from jax import lax
from jax.experimental import pallas as pl
from jax.experimental.pallas import tpu as pltpu
from jax.experimental.pallas import tpu_sc as plsc

N = 10000
E = 320000
NDF = 128
NH = 128

NC = 2
NS = 16
NW = NC * NS
EPW = E // NW
K = 80
NCH = EPW // K
NBUF = 2
RPS = 624
TAIL = N - NS * RPS
ZR = 16
DEGW = 128

_mesh = plsc.VectorSubcoreMesh(
    core_axis_name="c", subcore_axis_name="s", num_cores=NC, num_subcores=NS)


def _zero_rows(ref, nrows, ncols):
    z = jnp.zeros((16,), jnp.float32)

    @pl.loop(0, nrows)
    def _(r):
        for k in range(ncols // 16):
            ref[r, pl.ds(k * 16, 16)] = z


def _zero_acc_slice(zbuf, acc, s):
    base = s * RPS
    for j in range(RPS // ZR):
        pltpu.sync_copy(zbuf, acc.at[pl.ds(base + j * ZR, ZR)])

    @pl.when(s == NS - 1)
    def _():
        pltpu.sync_copy(zbuf.at[pl.ds(0, TAIL)], acc.at[pl.ds(NS * RPS, TAIL)])


def _acc_to_out(acc, out_hbm, c, s):
    base = s * RPS
    pltpu.sync_copy(acc.at[pl.ds(base, RPS)], out_hbm.at[c, pl.ds(base, RPS)])

    @pl.when(s == NS - 1)
    def _():
        pltpu.sync_copy(acc.at[pl.ds(NS * RPS, TAIL)],
                        out_hbm.at[c, pl.ds(NS * RPS, TAIL)])


@functools.partial(
    pl.kernel,
    out_type=jax.ShapeDtypeStruct((NC, N, DEGW), jnp.float32),
    mesh=_mesh,
    scratch_types=[
        pltpu.VMEM((NCH, K), jnp.int32),
        pltpu.VMEM((K, DEGW), jnp.float32),
        pltpu.VMEM((ZR, DEGW), jnp.float32),
        pltpu.VMEM_SHARED((N, DEGW), jnp.float32),
    ],
)
def _deg(dst_hbm, out_hbm, dst_v, ones_v, zbuf, acc):
    c = lax.axis_index("c")
    s = lax.axis_index("s")
    w = s * NC + c

    one = jnp.ones((16,), jnp.float32)

    @pl.loop(0, K)
    def _(r):
        for k in range(DEGW // 16):
            ones_v[r, pl.ds(k * 16, 16)] = one

    _zero_rows(zbuf, ZR, DEGW)
    _zero_acc_slice(zbuf, acc, s)
    plsc.subcore_barrier()

    pltpu.sync_copy(dst_hbm.at[w], dst_v)

    @pl.loop(0, NCH)
    def _(j):
        pltpu.sync_copy(ones_v, acc.at[dst_v.at[j]], add=True)

    plsc.subcore_barrier()
    _acc_to_out(acc, out_hbm, c, s)


@functools.partial(
    pl.kernel,
    out_type=jax.ShapeDtypeStruct((NC, N, NH), jnp.float32),
    mesh=_mesh,
    scratch_types=[
        pltpu.VMEM((EPW,), jnp.int32),
        pltpu.VMEM((NCH, K), jnp.int32),
        pltpu.VMEM((K, NH), jnp.float32),
        pltpu.VMEM((K, NH), jnp.float32),
        pltpu.VMEM((ZR, NH), jnp.float32),
        pltpu.SemaphoreType.DMA,
        pltpu.SemaphoreType.DMA,
        pltpu.VMEM_SHARED((N, NH), jnp.float32),
    ],
)
def _agg(y_hbm, src_hbm, dst_hbm, out_hbm, src_v, dst_v, rows_a, rows_b,
         zbuf, sem_a, sem_b, acc):
    c = lax.axis_index("c")
    s = lax.axis_index("s")
    w = s * NC + c

    _zero_rows(zbuf, ZR, NH)
    _zero_acc_slice(zbuf, acc, s)
    plsc.subcore_barrier()

    pltpu.sync_copy(src_hbm.at[w], src_v)
    pltpu.sync_copy(dst_hbm.at[w], dst_v)

    rows = [rows_a, rows_b]
    sems = [sem_a, sem_b]

    def _gather(j, b):
        pltpu.async_copy(y_hbm.at[src_v.at[pl.ds(j * K, K)]], rows[b],
                         sems[b])

    def _drain_scatter(j, b):
        pltpu.make_async_copy(y_hbm.at[src_v.at[pl.ds(j * K, K)]], rows[b],
                              sems[b]).wait()
        pltpu.sync_copy(rows[b], acc.at[dst_v.at[j]], add=True)

    for b in range(NBUF):
        _gather(b, b)

    @pl.loop(0, NCH // NBUF)
    def _(i):
        for b in range(NBUF):
            j = i * NBUF + b
            _drain_scatter(j, b)

            @pl.when(j + NBUF < NCH)
            def _():
                _gather(j + NBUF, b)

    for t in range(NCH - NCH % NBUF, NCH):
        _drain_scatter(t, t % NBUF)

    plsc.subcore_barrier()
    _acc_to_out(acc, out_hbm, c, s)


R = 1000
G = N // R


def _dinv_of(degp):
    deg = degp[0, :, 0:1] + degp[1, :, 0:1] + 1.0
    return lax.rsqrt(jnp.maximum(deg, 1.0))


def _pre_body(dx, cx, Wd, bd, Wc, bc, Wg0a, Wg0b, bg0, W1, y1raw):
    f32 = jnp.float32
    x_d = jnp.maximum(jnp.dot(dx[...], Wd[...], preferred_element_type=f32)
                      + bd[...], 0.0)
    hc = jnp.dot(cx[...], Wc[...], preferred_element_type=f32) + bc[...]
    x_c = jnp.maximum(jnp.concatenate([jnp.sin(hc), jnp.cos(hc)], axis=-1), 0.0)
    xg = jnp.maximum(jnp.dot(x_d, Wg0a[...], preferred_element_type=f32)
                     + jnp.dot(x_c, Wg0b[...], preferred_element_type=f32)
                     + bg0[...], 0.0)
    y1raw[...] = jnp.dot(xg, W1[...], preferred_element_type=f32)


def _scale_body(yraw, degp, y):
    y[...] = yraw[...] * _dinv_of(degp)


def _mid_body(p, yprev, degp, b, W, ynext):
    dinv = _dinv_of(degp)
    xg = jnp.maximum(dinv * (p[0] + p[1] + yprev[...]) + b[...], 0.0)
    ynext[...] = jnp.dot(xg, W[...], preferred_element_type=jnp.float32) * dinv


def _post_body(p, yprev, degp, b, Wp1, bp1, Wp2, bp2, out):
    f32 = jnp.float32
    dinv = _dinv_of(degp)
    xg = jnp.maximum(dinv * (p[0] + p[1] + yprev[...]) + b[...], 0.0)
    hid = jnp.maximum(jnp.dot(xg, Wp1[...], preferred_element_type=f32)
                      + bp1[...], 0.0)
    out[...] = jnp.dot(hid, Wp2[...], preferred_element_type=f32) + bp2[...]


def _full(shape):
    return pl.BlockSpec(shape, lambda i: tuple(0 for _ in shape))


def _rows(width):
    return pl.BlockSpec((R, width), lambda i: (i, 0))


_degp_spec = pl.BlockSpec((2, R, DEGW), lambda i: (0, i, 0))
_part_spec = pl.BlockSpec((2, R, NH), lambda i: (0, i, 0))


def kernel(discrete_x, continous_x, edge_index, W_d, b_d, W_c, b_c, W_g0,
           b_g0, W1, b1, W2, b2, Wp1, bp1, Wp2, bp2):
    src = edge_index[0].reshape(NW, EPW)
    dst = edge_index[1].reshape(NW, NCH, K)

    degp = _deg(dst)

    y1raw = pl.pallas_call(
        _pre_body,
        grid=(G,),
        in_specs=[_rows(NDF), _rows(16),
                  _full((NDF, NDF)), _full((1, NDF)),
                  _full((16, 64)), _full((1, 64)),
                  _full((NDF, NH)), _full((NDF, NH)), _full((1, NH)),
                  _full((NH, NH))],
        out_specs=_rows(NH),
        out_shape=jax.ShapeDtypeStruct((N, NH), jnp.float32),
    )(discrete_x, continous_x, W_d, b_d.reshape(1, -1),
      W_c, b_c.reshape(1, -1), W_g0[:NDF], W_g0[NDF:], b_g0.reshape(1, -1),
      W1)

    y1 = pl.pallas_call(
        _scale_body,
        grid=(G,),
        in_specs=[_rows(NH), _degp_spec],
        out_specs=_rows(NH),
        out_shape=jax.ShapeDtypeStruct((N, NH), jnp.float32),
    )(y1raw, degp)

    p1 = _agg(y1, src, dst)

    y2 = pl.pallas_call(
        _mid_body,
        grid=(G,),
        in_specs=[_part_spec, _rows(NH), _degp_spec,
                  _full((1, NH)), _full((NH, NH))],
        out_specs=_rows(NH),
        out_shape=jax.ShapeDtypeStruct((N, NH), jnp.float32),
    )(p1, y1, degp, b1.reshape(1, -1), W2)

    p2 = _agg(y2, src, dst)

    pred = pl.pallas_call(
        _post_body,
        grid=(G,),
        in_specs=[_part_spec, _rows(NH), _degp_spec,
                  _full((1, NH)), _full((NH, NH)), _full((1, NH)),
                  _full((NH, 1)), _full((1, 1))],
        out_specs=_rows(1),
        out_shape=jax.ShapeDtypeStruct((N, 1), jnp.float32),
    )(p2, y2, degp, b2.reshape(1, -1), Wp1, bp1.reshape(1, -1),
      Wp2, bp2.reshape(1, 1))

    return pred


# speedup vs baseline: 1.1382x; 1.1382x over previous
import functools

import jax
import jax.numpy as jnp
from jax import lax
from jax.experimental import pallas as pl
from jax.experimental.pallas import tpu as pltpu
from jax.experimental.pallas import tpu_sc as plsc

N = 10000
E = 320000
NDF = 128
NH = 128

NC = 2
NS = 16
NW = NC * NS
EPW = E // NW
K = 80
NCH = EPW // K
NBUF = 3
RPS = 624
TAIL = N - NS * RPS
ZR = 16
DEGW = 128

_mesh = plsc.VectorSubcoreMesh(
    core_axis_name="c", subcore_axis_name="s", num_cores=NC, num_subcores=NS)


def _zero_rows(ref, nrows, ncols):
    z = jnp.zeros((16,), jnp.float32)

    @pl.loop(0, nrows)
    def _(r):
        for k in range(ncols // 16):
            ref[r, pl.ds(k * 16, 16)] = z


def _zero_acc_slice(zbuf, acc, s):
    base = s * RPS
    for j in range(RPS // ZR):
        pltpu.sync_copy(zbuf, acc.at[pl.ds(base + j * ZR, ZR)])

    @pl.when(s == NS - 1)
    def _():
        pltpu.sync_copy(zbuf.at[pl.ds(0, TAIL)], acc.at[pl.ds(NS * RPS, TAIL)])


def _acc_to_out(acc, out_hbm, c, s):
    base = s * RPS
    pltpu.sync_copy(acc.at[pl.ds(base, RPS)], out_hbm.at[c, pl.ds(base, RPS)])

    @pl.when(s == NS - 1)
    def _():
        pltpu.sync_copy(acc.at[pl.ds(NS * RPS, TAIL)],
                        out_hbm.at[c, pl.ds(NS * RPS, TAIL)])


@functools.partial(
    pl.kernel,
    out_type=jax.ShapeDtypeStruct((NC, N, DEGW), jnp.float32),
    mesh=_mesh,
    scratch_types=[
        pltpu.VMEM((NCH, K), jnp.int32),
        pltpu.VMEM((K, DEGW), jnp.float32),
        pltpu.VMEM((ZR, DEGW), jnp.float32),
        pltpu.VMEM_SHARED((N, DEGW), jnp.float32),
    ],
)
def _deg(dst_hbm, out_hbm, dst_v, ones_v, zbuf, acc):
    c = lax.axis_index("c")
    s = lax.axis_index("s")
    w = s * NC + c

    one = jnp.ones((16,), jnp.float32)

    @pl.loop(0, K)
    def _(r):
        for k in range(DEGW // 16):
            ones_v[r, pl.ds(k * 16, 16)] = one

    _zero_rows(zbuf, ZR, DEGW)
    _zero_acc_slice(zbuf, acc, s)
    plsc.subcore_barrier()

    pltpu.sync_copy(dst_hbm.at[w], dst_v)

    @pl.loop(0, NCH)
    def _(j):
        pltpu.sync_copy(ones_v, acc.at[dst_v.at[j]], add=True)

    plsc.subcore_barrier()
    _acc_to_out(acc, out_hbm, c, s)


@functools.partial(
    pl.kernel,
    out_type=jax.ShapeDtypeStruct((NC, N, NH), jnp.float32),
    mesh=_mesh,
    scratch_types=[
        pltpu.VMEM((EPW,), jnp.int32),
        pltpu.VMEM((NBUF, K), jnp.int32),
        pltpu.VMEM((K, NH), jnp.float32),
        pltpu.VMEM((K, NH), jnp.float32),
        pltpu.VMEM((K, NH), jnp.float32),
        pltpu.VMEM((ZR, NH), jnp.float32),
        pltpu.SemaphoreType.DMA,
        pltpu.SemaphoreType.DMA,
        pltpu.SemaphoreType.DMA,
        pltpu.SemaphoreType.DMA,
        pltpu.SemaphoreType.DMA,
        pltpu.SemaphoreType.DMA,
        pltpu.VMEM_SHARED((N, NH), jnp.float32),
    ],
)
def _agg(y_hbm, src_hbm, dst_hbm, out_hbm, src_v, dstring, rows_a, rows_b,
         rows_c, zbuf, sem_a, sem_b, sem_c, semd_a, semd_b, semd_c, acc):
    c = lax.axis_index("c")
    s = lax.axis_index("s")
    w = s * NC + c

    _zero_rows(zbuf, ZR, NH)
    _zero_acc_slice(zbuf, acc, s)
    plsc.subcore_barrier()

    pltpu.sync_copy(src_hbm.at[w], src_v)

    rows = [rows_a, rows_b, rows_c]
    sems = [sem_a, sem_b, sem_c]
    semd = [semd_a, semd_b, semd_c]

    def _gather(j, b):
        pltpu.async_copy(y_hbm.at[src_v.at[pl.ds(j * K, K)]], rows[b],
                         sems[b])
        pltpu.async_copy(dst_hbm.at[w, pl.ds(j, 1)], dstring.at[pl.ds(b, 1)],
                         semd[b])

    def _drain_scatter(j, b):
        pltpu.make_async_copy(y_hbm.at[src_v.at[pl.ds(j * K, K)]], rows[b],
                              sems[b]).wait()
        pltpu.make_async_copy(dst_hbm.at[w, pl.ds(j, 1)],
                              dstring.at[pl.ds(b, 1)], semd[b]).wait()
        pltpu.sync_copy(rows[b], acc.at[dstring.at[b]], add=True)

    for b in range(NBUF):
        _gather(b, b)

    @pl.loop(0, NCH // NBUF)
    def _(i):
        for b in range(NBUF):
            j = i * NBUF + b
            _drain_scatter(j, b)

            @pl.when(j + NBUF < NCH)
            def _():
                _gather(j + NBUF, b)

    for t in range(NCH - NCH % NBUF, NCH):
        _drain_scatter(t, t % NBUF)

    plsc.subcore_barrier()
    _acc_to_out(acc, out_hbm, c, s)


R = 1000
G = N // R


def _dinv_of(degp):
    deg = degp[0, :, 0:1] + degp[1, :, 0:1] + 1.0
    return lax.rsqrt(jnp.maximum(deg, 1.0))


def _pre_body(dx, cx, Wd, bd, Wc, bc, Wg0a, Wg0b, bg0, W1, y1raw):
    f32 = jnp.float32
    x_d = jnp.maximum(jnp.dot(dx[...], Wd[...], preferred_element_type=f32)
                      + bd[...], 0.0)
    hc = jnp.dot(cx[...], Wc[...], preferred_element_type=f32) + bc[...]
    x_c = jnp.maximum(jnp.concatenate([jnp.sin(hc), jnp.cos(hc)], axis=-1), 0.0)
    xg = jnp.maximum(jnp.dot(x_d, Wg0a[...], preferred_element_type=f32)
                     + jnp.dot(x_c, Wg0b[...], preferred_element_type=f32)
                     + bg0[...], 0.0)
    y1raw[...] = jnp.dot(xg, W1[...], preferred_element_type=f32)


def _scale_body(yraw, degp, y):
    y[...] = yraw[...] * _dinv_of(degp)


def _mid_body(p, yprev, degp, b, W, ynext):
    dinv = _dinv_of(degp)
    xg = jnp.maximum(dinv * (p[0] + p[1] + yprev[...]) + b[...], 0.0)
    ynext[...] = jnp.dot(xg, W[...], preferred_element_type=jnp.float32) * dinv


def _post_body(p, yprev, degp, b, Wp1, bp1, Wp2, bp2, out):
    f32 = jnp.float32
    dinv = _dinv_of(degp)
    xg = jnp.maximum(dinv * (p[0] + p[1] + yprev[...]) + b[...], 0.0)
    hid = jnp.maximum(jnp.dot(xg, Wp1[...], preferred_element_type=f32)
                      + bp1[...], 0.0)
    out[...] = jnp.dot(hid, Wp2[...], preferred_element_type=f32) + bp2[...]


def _full(shape):
    return pl.BlockSpec(shape, lambda i: tuple(0 for _ in shape))


def _rows(width):
    return pl.BlockSpec((R, width), lambda i: (i, 0))


_degp_spec = pl.BlockSpec((2, R, DEGW), lambda i: (0, i, 0))
_part_spec = pl.BlockSpec((2, R, NH), lambda i: (0, i, 0))


def kernel(discrete_x, continous_x, edge_index, W_d, b_d, W_c, b_c, W_g0,
           b_g0, W1, b1, W2, b2, Wp1, bp1, Wp2, bp2):
    src = edge_index[0].reshape(NW, EPW)
    dst = edge_index[1].reshape(NW, NCH, K)

    degp = _deg(dst)

    y1raw = pl.pallas_call(
        _pre_body,
        grid=(G,),
        in_specs=[_rows(NDF), _rows(16),
                  _full((NDF, NDF)), _full((1, NDF)),
                  _full((16, 64)), _full((1, 64)),
                  _full((NDF, NH)), _full((NDF, NH)), _full((1, NH)),
                  _full((NH, NH))],
        out_specs=_rows(NH),
        out_shape=jax.ShapeDtypeStruct((N, NH), jnp.float32),
    )(discrete_x, continous_x, W_d, b_d.reshape(1, -1),
      W_c, b_c.reshape(1, -1), W_g0[:NDF], W_g0[NDF:], b_g0.reshape(1, -1),
      W1)

    y1 = pl.pallas_call(
        _scale_body,
        grid=(G,),
        in_specs=[_rows(NH), _degp_spec],
        out_specs=_rows(NH),
        out_shape=jax.ShapeDtypeStruct((N, NH), jnp.float32),
    )(y1raw, degp)

    p1 = _agg(y1, src, dst)

    y2 = pl.pallas_call(
        _mid_body,
        grid=(G,),
        in_specs=[_part_spec, _rows(NH), _degp_spec,
                  _full((1, NH)), _full((NH, NH))],
        out_specs=_rows(NH),
        out_shape=jax.ShapeDtypeStruct((N, NH), jnp.float32),
    )(p1, y1, degp, b1.reshape(1, -1), W2)

    p2 = _agg(y2, src, dst)

    pred = pl.pallas_call(
        _post_body,
        grid=(G,),
        in_specs=[_part_spec, _rows(NH), _degp_spec,
                  _full((1, NH)), _full((NH, NH)), _full((1, NH)),
                  _full((NH, 1)), _full((1, 1))],
        out_specs=_rows(1),
        out_shape=jax.ShapeDtypeStruct((N, 1), jnp.float32),
    )(p2, y2, degp, b2.reshape(1, -1), Wp1, bp1.reshape(1, -1),
      Wp2, bp2.reshape(1, 1))

    return pred


# speedup vs baseline: 1.1655x; 1.0240x over previous
import functools

import jax
import jax.numpy as jnp
from jax import lax
from jax.experimental import pallas as pl
from jax.experimental.pallas import tpu as pltpu
from jax.experimental.pallas import tpu_sc as plsc

N = 10000
E = 320000
NDF = 128
NH = 128

NC = 2
NS = 16
NW = NC * NS
EPW = E // NW
K = 80
NCH = EPW // K
NBUF = 4
RPS = 624
TAIL = N - NS * RPS
ZR = 16
DEGW = 128

_mesh = plsc.VectorSubcoreMesh(
    core_axis_name="c", subcore_axis_name="s", num_cores=NC, num_subcores=NS)


def _zero_rows(ref, nrows, ncols):
    z = jnp.zeros((16,), jnp.float32)

    @pl.loop(0, nrows)
    def _(r):
        for k in range(ncols // 16):
            ref[r, pl.ds(k * 16, 16)] = z


def _zero_acc_slice(zbuf, acc, s):
    base = s * RPS
    for j in range(RPS // ZR):
        pltpu.sync_copy(zbuf, acc.at[pl.ds(base + j * ZR, ZR)])

    @pl.when(s == NS - 1)
    def _():
        pltpu.sync_copy(zbuf.at[pl.ds(0, TAIL)], acc.at[pl.ds(NS * RPS, TAIL)])


def _acc_to_out(acc, out_hbm, c, s):
    base = s * RPS
    pltpu.sync_copy(acc.at[pl.ds(base, RPS)], out_hbm.at[c, pl.ds(base, RPS)])

    @pl.when(s == NS - 1)
    def _():
        pltpu.sync_copy(acc.at[pl.ds(NS * RPS, TAIL)],
                        out_hbm.at[c, pl.ds(NS * RPS, TAIL)])


@functools.partial(
    pl.kernel,
    out_type=jax.ShapeDtypeStruct((NC, N, DEGW), jnp.float32),
    mesh=_mesh,
    scratch_types=[
        pltpu.VMEM((NCH, K), jnp.int32),
        pltpu.VMEM((K, DEGW), jnp.float32),
        pltpu.VMEM((ZR, DEGW), jnp.float32),
        pltpu.VMEM_SHARED((N, DEGW), jnp.float32),
    ],
)
def _deg(dst_hbm, out_hbm, dst_v, ones_v, zbuf, acc):
    c = lax.axis_index("c")
    s = lax.axis_index("s")
    w = s * NC + c

    one = jnp.ones((16,), jnp.float32)

    @pl.loop(0, K)
    def _(r):
        for k in range(DEGW // 16):
            ones_v[r, pl.ds(k * 16, 16)] = one

    _zero_rows(zbuf, ZR, DEGW)
    _zero_acc_slice(zbuf, acc, s)
    plsc.subcore_barrier()

    pltpu.sync_copy(dst_hbm.at[w], dst_v)

    @pl.loop(0, NCH)
    def _(j):
        pltpu.sync_copy(ones_v, acc.at[dst_v.at[j]], add=True)

    plsc.subcore_barrier()
    _acc_to_out(acc, out_hbm, c, s)


@functools.partial(
    pl.kernel,
    out_type=jax.ShapeDtypeStruct((NC, N, NH), jnp.float32),
    mesh=_mesh,
    scratch_types=[
        pltpu.VMEM((2 * NBUF, K), jnp.int32),
        pltpu.VMEM((2 * NBUF, K), jnp.int32),
        pltpu.VMEM((K, NH), jnp.float32),
        pltpu.VMEM((K, NH), jnp.float32),
        pltpu.VMEM((K, NH), jnp.float32),
        pltpu.VMEM((K, NH), jnp.float32),
        pltpu.VMEM((ZR, NH), jnp.float32),
        pltpu.SemaphoreType.DMA,
        pltpu.SemaphoreType.DMA,
        pltpu.SemaphoreType.DMA,
        pltpu.SemaphoreType.DMA,
        pltpu.SemaphoreType.DMA,
        pltpu.SemaphoreType.DMA,
        pltpu.SemaphoreType.DMA,
        pltpu.SemaphoreType.DMA,
        pltpu.SemaphoreType.DMA,
        pltpu.SemaphoreType.DMA,
        pltpu.SemaphoreType.DMA,
        pltpu.SemaphoreType.DMA,
        pltpu.VMEM_SHARED((N, NH), jnp.float32),
    ],
)
def _agg(y_hbm, src_hbm, dst_hbm, out_hbm, srcring, dstring, rows_a, rows_b,
         rows_c, rows_d, zbuf, sem_a, sem_b, sem_c, sem_d, semi_0, semi_1,
         semi_2, semi_3, semi_4, semi_5, semi_6, semi_7, acc):
    c = lax.axis_index("c")
    s = lax.axis_index("s")
    w = s * NC + c
    SI = 2 * NBUF

    _zero_rows(zbuf, ZR, NH)
    _zero_acc_slice(zbuf, acc, s)
    plsc.subcore_barrier()

    rows = [rows_a, rows_b, rows_c, rows_d]
    sems = [sem_a, sem_b, sem_c, sem_d]
    semi = [semi_0, semi_1, semi_2, semi_3, semi_4, semi_5, semi_6, semi_7]

    def _fetch_idx(j, q):
        pltpu.async_copy(src_hbm.at[w, pl.ds(j, 1)], srcring.at[pl.ds(q, 1)],
                         semi[q])
        pltpu.async_copy(dst_hbm.at[w, pl.ds(j, 1)], dstring.at[pl.ds(q, 1)],
                         semi[q])

    def _gather(j, b, q):
        pltpu.make_async_copy(src_hbm.at[w, pl.ds(j, 1)],
                              srcring.at[pl.ds(q, 1)], semi[q]).wait()
        pltpu.make_async_copy(dst_hbm.at[w, pl.ds(j, 1)],
                              dstring.at[pl.ds(q, 1)], semi[q]).wait()
        pltpu.async_copy(y_hbm.at[srcring.at[q]], rows[b], sems[b])

    def _drain_scatter(j, b, q):
        pltpu.make_async_copy(y_hbm.at[srcring.at[q]], rows[b],
                              sems[b]).wait()
        pltpu.sync_copy(rows[b], acc.at[dstring.at[q]], add=True)

        @pl.when(j + SI < NCH)
        def _():
            _fetch_idx(j + SI, q)

    for q in range(SI):
        _fetch_idx(q, q)
    for b in range(NBUF):
        _gather(b, b, b)

    @pl.loop(0, NCH // SI)
    def _(i):
        for u in range(SI):
            j = i * SI + u
            b = u % NBUF
            _drain_scatter(j, b, u)

            @pl.when(j + NBUF < NCH)
            def _():
                _gather(j + NBUF, b, (u + NBUF) % SI)

    for t in range(SI * (NCH // SI), NCH):
        _drain_scatter(t, t % NBUF, t % SI)
        if t + NBUF < NCH:
            _gather(t + NBUF, (t + NBUF) % NBUF, (t + NBUF) % SI)

    plsc.subcore_barrier()
    _acc_to_out(acc, out_hbm, c, s)


R = 1000
G = N // R


def _dinv_of(degp):
    deg = degp[0, :, 0:1] + degp[1, :, 0:1] + 1.0
    return lax.rsqrt(jnp.maximum(deg, 1.0))


def _pre_body(dx, cx, Wd, bd, Wc, bc, Wg0a, Wg0b, bg0, W1, y1raw):
    f32 = jnp.float32
    x_d = jnp.maximum(jnp.dot(dx[...], Wd[...], preferred_element_type=f32)
                      + bd[...], 0.0)
    hc = jnp.dot(cx[...], Wc[...], preferred_element_type=f32) + bc[...]
    x_c = jnp.maximum(jnp.concatenate([jnp.sin(hc), jnp.cos(hc)], axis=-1), 0.0)
    xg = jnp.maximum(jnp.dot(x_d, Wg0a[...], preferred_element_type=f32)
                     + jnp.dot(x_c, Wg0b[...], preferred_element_type=f32)
                     + bg0[...], 0.0)
    y1raw[...] = jnp.dot(xg, W1[...], preferred_element_type=f32)


def _scale_body(yraw, degp, y):
    y[...] = yraw[...] * _dinv_of(degp)


def _mid_body(p, yprev, degp, b, W, ynext):
    dinv = _dinv_of(degp)
    xg = jnp.maximum(dinv * (p[0] + p[1] + yprev[...]) + b[...], 0.0)
    ynext[...] = jnp.dot(xg, W[...], preferred_element_type=jnp.float32) * dinv


def _post_body(p, yprev, degp, b, Wp1, bp1, Wp2, bp2, out):
    f32 = jnp.float32
    dinv = _dinv_of(degp)
    xg = jnp.maximum(dinv * (p[0] + p[1] + yprev[...]) + b[...], 0.0)
    hid = jnp.maximum(jnp.dot(xg, Wp1[...], preferred_element_type=f32)
                      + bp1[...], 0.0)
    out[...] = jnp.dot(hid, Wp2[...], preferred_element_type=f32) + bp2[...]


def _full(shape):
    return pl.BlockSpec(shape, lambda i: tuple(0 for _ in shape))


def _rows(width):
    return pl.BlockSpec((R, width), lambda i: (i, 0))


_degp_spec = pl.BlockSpec((2, R, DEGW), lambda i: (0, i, 0))
_part_spec = pl.BlockSpec((2, R, NH), lambda i: (0, i, 0))


def kernel(discrete_x, continous_x, edge_index, W_d, b_d, W_c, b_c, W_g0,
           b_g0, W1, b1, W2, b2, Wp1, bp1, Wp2, bp2):
    src = edge_index[0].reshape(NW, NCH, K)
    dst = edge_index[1].reshape(NW, NCH, K)

    degp = _deg(dst)

    y1raw = pl.pallas_call(
        _pre_body,
        grid=(G,),
        in_specs=[_rows(NDF), _rows(16),
                  _full((NDF, NDF)), _full((1, NDF)),
                  _full((16, 64)), _full((1, 64)),
                  _full((NDF, NH)), _full((NDF, NH)), _full((1, NH)),
                  _full((NH, NH))],
        out_specs=_rows(NH),
        out_shape=jax.ShapeDtypeStruct((N, NH), jnp.float32),
    )(discrete_x, continous_x, W_d, b_d.reshape(1, -1),
      W_c, b_c.reshape(1, -1), W_g0[:NDF], W_g0[NDF:], b_g0.reshape(1, -1),
      W1)

    y1 = pl.pallas_call(
        _scale_body,
        grid=(G,),
        in_specs=[_rows(NH), _degp_spec],
        out_specs=_rows(NH),
        out_shape=jax.ShapeDtypeStruct((N, NH), jnp.float32),
    )(y1raw, degp)

    p1 = _agg(y1, src, dst)

    y2 = pl.pallas_call(
        _mid_body,
        grid=(G,),
        in_specs=[_part_spec, _rows(NH), _degp_spec,
                  _full((1, NH)), _full((NH, NH))],
        out_specs=_rows(NH),
        out_shape=jax.ShapeDtypeStruct((N, NH), jnp.float32),
    )(p1, y1, degp, b1.reshape(1, -1), W2)

    p2 = _agg(y2, src, dst)

    pred = pl.pallas_call(
        _post_body,
        grid=(G,),
        in_specs=[_part_spec, _rows(NH), _degp_spec,
                  _full((1, NH)), _full((NH, NH)), _full((1, NH)),
                  _full((NH, 1)), _full((1, 1))],
        out_specs=_rows(1),
        out_shape=jax.ShapeDtypeStruct((N, 1), jnp.float32),
    )(p2, y2, degp, b2.reshape(1, -1), Wp1, bp1.reshape(1, -1),
      Wp2, bp2.reshape(1, 1))

    return pred


# speedup vs baseline: 1.1694x; 1.0033x over previous
import functools

import jax
import jax.numpy as jnp
from jax import lax
from jax.experimental import pallas as pl
from jax.experimental.pallas import tpu as pltpu
from jax.experimental.pallas import tpu_sc as plsc

N = 10000
E = 320000
NDF = 128
NH = 128

NC = 2
NS = 16
NW = NC * NS
EPW = E // NW
K = 80
NCH = EPW // K
NBUF = 4
RPS = 624
TAIL = N - NS * RPS
ZR = 16
DEGW = 128

_mesh = plsc.VectorSubcoreMesh(
    core_axis_name="c", subcore_axis_name="s", num_cores=NC, num_subcores=NS)


def _zero_rows(ref, nrows, ncols):
    z = jnp.zeros((16,), jnp.float32)

    @pl.loop(0, nrows)
    def _(r):
        for k in range(ncols // 16):
            ref[r, pl.ds(k * 16, 16)] = z


def _zero_acc_slice(zbuf, acc, s):
    base = s * RPS
    for j in range(RPS // ZR):
        pltpu.sync_copy(zbuf, acc.at[pl.ds(base + j * ZR, ZR)])

    @pl.when(s == NS - 1)
    def _():
        pltpu.sync_copy(zbuf.at[pl.ds(0, TAIL)], acc.at[pl.ds(NS * RPS, TAIL)])


def _acc_to_out(acc, out_hbm, c, s):
    base = s * RPS
    pltpu.sync_copy(acc.at[pl.ds(base, RPS)], out_hbm.at[c, pl.ds(base, RPS)])

    @pl.when(s == NS - 1)
    def _():
        pltpu.sync_copy(acc.at[pl.ds(NS * RPS, TAIL)],
                        out_hbm.at[c, pl.ds(NS * RPS, TAIL)])


@functools.partial(
    pl.kernel,
    out_type=jax.ShapeDtypeStruct((NC, N, DEGW), jnp.float32),
    mesh=_mesh,
    scratch_types=[
        pltpu.VMEM((NCH, K), jnp.int32),
        pltpu.VMEM((K, DEGW), jnp.float32),
        pltpu.VMEM((ZR, DEGW), jnp.float32),
        pltpu.SemaphoreType.DMA,
        pltpu.SemaphoreType.DMA,
        pltpu.SemaphoreType.DMA,
        pltpu.SemaphoreType.DMA,
        pltpu.SemaphoreType.DMA,
        pltpu.SemaphoreType.DMA,
        pltpu.SemaphoreType.DMA,
        pltpu.SemaphoreType.DMA,
        pltpu.VMEM_SHARED((N, DEGW), jnp.float32),
    ],
)
def _deg(dst_hbm, out_hbm, dst_v, ones_v, zbuf, sem_0, sem_1, sem_2, sem_3,
         sem_4, sem_5, sem_6, sem_7, acc):
    c = lax.axis_index("c")
    s = lax.axis_index("s")
    w = s * NC + c

    one = jnp.ones((16,), jnp.float32)

    @pl.loop(0, K)
    def _(r):
        for k in range(DEGW // 16):
            ones_v[r, pl.ds(k * 16, 16)] = one

    _zero_rows(zbuf, ZR, DEGW)
    _zero_acc_slice(zbuf, acc, s)
    plsc.subcore_barrier()

    pltpu.sync_copy(dst_hbm.at[w], dst_v)

    sems = [sem_0, sem_1, sem_2, sem_3, sem_4, sem_5, sem_6, sem_7]
    NQ = len(sems)

    def _scat(j, q):
        pltpu.async_copy(ones_v, acc.at[dst_v.at[j]], sems[q], add=True)

    def _wait(j, q):
        pltpu.make_async_copy(ones_v, acc.at[dst_v.at[j]], sems[q]).wait()

    for q in range(NQ):
        _scat(q, q)

    @pl.loop(0, NCH // NQ)
    def _(i):
        for u in range(NQ):
            j = i * NQ + u
            _wait(j, u)

            @pl.when(j + NQ < NCH)
            def _():
                _scat(j + NQ, u)

    for t in range(NQ * (NCH // NQ), NCH):
        _wait(t, t % NQ)

    plsc.subcore_barrier()
    _acc_to_out(acc, out_hbm, c, s)


@functools.partial(
    pl.kernel,
    out_type=jax.ShapeDtypeStruct((NC, N, NH), jnp.float32),
    mesh=_mesh,
    scratch_types=[
        pltpu.VMEM((2 * NBUF, K), jnp.int32),
        pltpu.VMEM((2 * NBUF, K), jnp.int32),
        pltpu.VMEM((K, NH), jnp.float32),
        pltpu.VMEM((K, NH), jnp.float32),
        pltpu.VMEM((K, NH), jnp.float32),
        pltpu.VMEM((K, NH), jnp.float32),
        pltpu.VMEM((ZR, NH), jnp.float32),
        pltpu.SemaphoreType.DMA,
        pltpu.SemaphoreType.DMA,
        pltpu.SemaphoreType.DMA,
        pltpu.SemaphoreType.DMA,
        pltpu.SemaphoreType.DMA,
        pltpu.SemaphoreType.DMA,
        pltpu.SemaphoreType.DMA,
        pltpu.SemaphoreType.DMA,
        pltpu.SemaphoreType.DMA,
        pltpu.SemaphoreType.DMA,
        pltpu.SemaphoreType.DMA,
        pltpu.SemaphoreType.DMA,
        pltpu.VMEM_SHARED((N, NH), jnp.float32),
    ],
)
def _agg(y_hbm, src_hbm, dst_hbm, out_hbm, srcring, dstring, rows_a, rows_b,
         rows_c, rows_d, zbuf, sem_a, sem_b, sem_c, sem_d, semi_0, semi_1,
         semi_2, semi_3, semi_4, semi_5, semi_6, semi_7, acc):
    c = lax.axis_index("c")
    s = lax.axis_index("s")
    w = s * NC + c
    SI = 2 * NBUF

    _zero_rows(zbuf, ZR, NH)
    _zero_acc_slice(zbuf, acc, s)
    plsc.subcore_barrier()

    rows = [rows_a, rows_b, rows_c, rows_d]
    sems = [sem_a, sem_b, sem_c, sem_d]
    semi = [semi_0, semi_1, semi_2, semi_3, semi_4, semi_5, semi_6, semi_7]

    def _fetch_idx(j, q):
        pltpu.async_copy(src_hbm.at[w, pl.ds(j, 1)], srcring.at[pl.ds(q, 1)],
                         semi[q])
        pltpu.async_copy(dst_hbm.at[w, pl.ds(j, 1)], dstring.at[pl.ds(q, 1)],
                         semi[q])

    def _gather(j, b, q):
        pltpu.make_async_copy(src_hbm.at[w, pl.ds(j, 1)],
                              srcring.at[pl.ds(q, 1)], semi[q]).wait()
        pltpu.make_async_copy(dst_hbm.at[w, pl.ds(j, 1)],
                              dstring.at[pl.ds(q, 1)], semi[q]).wait()
        pltpu.async_copy(y_hbm.at[srcring.at[q]], rows[b], sems[b])

    def _drain_scatter(j, b, q):
        pltpu.make_async_copy(y_hbm.at[srcring.at[q]], rows[b],
                              sems[b]).wait()
        pltpu.sync_copy(rows[b], acc.at[dstring.at[q]], add=True)

        @pl.when(j + SI < NCH)
        def _():
            _fetch_idx(j + SI, q)

    for q in range(SI):
        _fetch_idx(q, q)
    for b in range(NBUF):
        _gather(b, b, b)

    @pl.loop(0, NCH // SI)
    def _(i):
        for u in range(SI):
            j = i * SI + u
            b = u % NBUF
            _drain_scatter(j, b, u)

            @pl.when(j + NBUF < NCH)
            def _():
                _gather(j + NBUF, b, (u + NBUF) % SI)

    for t in range(SI * (NCH // SI), NCH):
        _drain_scatter(t, t % NBUF, t % SI)
        if t + NBUF < NCH:
            _gather(t + NBUF, (t + NBUF) % NBUF, (t + NBUF) % SI)

    plsc.subcore_barrier()
    _acc_to_out(acc, out_hbm, c, s)


R = 1000
G = N // R


def _dinv_of(degp):
    deg = degp[0, :, 0:1] + degp[1, :, 0:1] + 1.0
    return lax.rsqrt(jnp.maximum(deg, 1.0))


def _pre_body(dx, cx, Wd, bd, Wc, bc, Wg0a, Wg0b, bg0, W1, y1raw):
    f32 = jnp.float32
    x_d = jnp.maximum(jnp.dot(dx[...], Wd[...], preferred_element_type=f32)
                      + bd[...], 0.0)
    hc = jnp.dot(cx[...], Wc[...], preferred_element_type=f32) + bc[...]
    x_c = jnp.maximum(jnp.concatenate([jnp.sin(hc), jnp.cos(hc)], axis=-1), 0.0)
    xg = jnp.maximum(jnp.dot(x_d, Wg0a[...], preferred_element_type=f32)
                     + jnp.dot(x_c, Wg0b[...], preferred_element_type=f32)
                     + bg0[...], 0.0)
    y1raw[...] = jnp.dot(xg, W1[...], preferred_element_type=f32)


def _scale_body(yraw, degp, y):
    y[...] = yraw[...] * _dinv_of(degp)


def _mid_body(p, yprev, degp, b, W, ynext):
    dinv = _dinv_of(degp)
    xg = jnp.maximum(dinv * (p[0] + p[1] + yprev[...]) + b[...], 0.0)
    ynext[...] = jnp.dot(xg, W[...], preferred_element_type=jnp.float32) * dinv


def _post_body(p, yprev, degp, b, Wp1, bp1, Wp2, bp2, out):
    f32 = jnp.float32
    dinv = _dinv_of(degp)
    xg = jnp.maximum(dinv * (p[0] + p[1] + yprev[...]) + b[...], 0.0)
    hid = jnp.maximum(jnp.dot(xg, Wp1[...], preferred_element_type=f32)
                      + bp1[...], 0.0)
    out[...] = jnp.dot(hid, Wp2[...], preferred_element_type=f32) + bp2[...]


def _full(shape):
    return pl.BlockSpec(shape, lambda i: tuple(0 for _ in shape))


def _rows(width):
    return pl.BlockSpec((R, width), lambda i: (i, 0))


_degp_spec = pl.BlockSpec((2, R, DEGW), lambda i: (0, i, 0))
_part_spec = pl.BlockSpec((2, R, NH), lambda i: (0, i, 0))


def kernel(discrete_x, continous_x, edge_index, W_d, b_d, W_c, b_c, W_g0,
           b_g0, W1, b1, W2, b2, Wp1, bp1, Wp2, bp2):
    src = edge_index[0].reshape(NW, NCH, K)
    dst = edge_index[1].reshape(NW, NCH, K)

    degp = _deg(dst)

    y1raw = pl.pallas_call(
        _pre_body,
        grid=(G,),
        in_specs=[_rows(NDF), _rows(16),
                  _full((NDF, NDF)), _full((1, NDF)),
                  _full((16, 64)), _full((1, 64)),
                  _full((NDF, NH)), _full((NDF, NH)), _full((1, NH)),
                  _full((NH, NH))],
        out_specs=_rows(NH),
        out_shape=jax.ShapeDtypeStruct((N, NH), jnp.float32),
    )(discrete_x, continous_x, W_d, b_d.reshape(1, -1),
      W_c, b_c.reshape(1, -1), W_g0[:NDF], W_g0[NDF:], b_g0.reshape(1, -1),
      W1)

    y1 = pl.pallas_call(
        _scale_body,
        grid=(G,),
        in_specs=[_rows(NH), _degp_spec],
        out_specs=_rows(NH),
        out_shape=jax.ShapeDtypeStruct((N, NH), jnp.float32),
    )(y1raw, degp)

    p1 = _agg(y1, src, dst)

    y2 = pl.pallas_call(
        _mid_body,
        grid=(G,),
        in_specs=[_part_spec, _rows(NH), _degp_spec,
                  _full((1, NH)), _full((NH, NH))],
        out_specs=_rows(NH),
        out_shape=jax.ShapeDtypeStruct((N, NH), jnp.float32),
    )(p1, y1, degp, b1.reshape(1, -1), W2)

    p2 = _agg(y2, src, dst)

    pred = pl.pallas_call(
        _post_body,
        grid=(G,),
        in_specs=[_part_spec, _rows(NH), _degp_spec,
                  _full((1, NH)), _full((NH, NH)), _full((1, NH)),
                  _full((NH, 1)), _full((1, 1))],
        out_specs=_rows(1),
        out_shape=jax.ShapeDtypeStruct((N, 1), jnp.float32),
    )(p2, y2, degp, b2.reshape(1, -1), Wp1, bp1.reshape(1, -1),
      Wp2, bp2.reshape(1, 1))

    return pred
